# Initial kernel scaffold; baseline (speedup 1.0000x reference)
import functools

import jax
import jax.numpy as jnp
from jax import lax
from jax.experimental import pallas as pl
from jax.experimental.pallas import tpu as pltpu
import jax.experimental.pallas.tpu_sc as plsc

B = 64
NUM = 1024
D = 64
H = 64
E = 16384
MAIN = 16

NS = 16
EPT = E // NS
WPT = (NUM * NUM) // NS
NCHUNK = 8
CHUNK = EPT // NCHUNK


def _sc_build_counts(src, dst, zeros):
    mesh = plsc.VectorSubcoreMesh(core_axis_name="c", subcore_axis_name="s")

    idx_scratch = [pltpu.VMEM((CHUNK,), jnp.int32) for _ in range(NCHUNK)]

    @functools.partial(
        pl.kernel,
        mesh=mesh,
        out_type=jax.ShapeDtypeStruct((NUM * NUM,), jnp.float32),
        scratch_types=[
            pltpu.VMEM((EPT,), jnp.int32),
            pltpu.VMEM((EPT,), jnp.int32),
            pltpu.VMEM((CHUNK,), jnp.float32),
            pltpu.VMEM_SHARED((NUM * NUM,), jnp.float32),
        ] + idx_scratch,
    )
    def body(src_hbm, dst_hbm, zero_hbm, out_hbm, ev_src, ev_dst, ones_v,
             acc, *idx_refs):
        cid = lax.axis_index("c")
        sid = lax.axis_index("s")
        base = sid * WPT
        pltpu.sync_copy(zero_hbm.at[pl.ds(base, WPT)], acc.at[pl.ds(base, WPT)])
        ebase = sid * EPT
        pltpu.sync_copy(src_hbm.at[pl.ds(ebase, EPT)], ev_src)
        pltpu.sync_copy(dst_hbm.at[pl.ds(ebase, EPT)], ev_dst)
        ones16 = jnp.ones((16,), jnp.float32)
        for k in range(CHUNK // 16):
            ones_v[pl.ds(k * 16, 16)] = ones16
        for j in range(NCHUNK):
            for k in range(CHUNK // 16):
                off = j * CHUNK + k * 16
                s16 = ev_src[pl.ds(off, 16)]
                d16 = ev_dst[pl.ds(off, 16)]
                idx_refs[j][pl.ds(k * 16, 16)] = s16 * NUM + d16
        plsc.subcore_barrier()
        for j in range(NCHUNK):
            pltpu.sync_copy(ones_v, acc.at[idx_refs[j]], add=True)
        plsc.subcore_barrier()

        @pl.when(cid == 0)
        def _():
            pltpu.sync_copy(acc.at[pl.ds(base, WPT)],
                            out_hbm.at[pl.ds(base, WPT)])

    return body(src, dst, zeros)


def _tc_gcn(x, embedding, W1, b1, W2, b2, W3, b3, at0):
    def body(x_ref, emb_ref, w1_ref, b1_ref, w2_ref, b2_ref, w3_ref, b3_ref,
             at_ref, out_ref, dinv_ref, ew1_ref):
        b = pl.program_id(0)

        @pl.when(b == 0)
        def _():
            at0_full = at_ref[...]
            deg = jnp.sum(at0_full, axis=0, keepdims=True)
            dinv_ref[...] = jnp.where(
                deg > 0, lax.rsqrt(jnp.maximum(deg, 1e-12)), 0.0)
            ew1_ref[...] = lax.dot_general(
                w1_ref[...], emb_ref[...], (((0,), (1,)), ((), ())),
                preferred_element_type=jnp.float32)

        dinv = dinv_ref[...]
        at = at_ref[...]
        xb = x_ref[...]

        x1 = ew1_ref[...] * xb
        a1 = jnp.dot(x1 * dinv, at, preferred_element_type=jnp.float32)
        h1 = jnp.maximum(a1 * dinv + b1_ref[...], 0.0)

        x2 = lax.dot_general(w2_ref[...], h1, (((0,), (0,)), ((), ())),
                             preferred_element_type=jnp.float32)
        a2 = jnp.dot(x2 * dinv, at, preferred_element_type=jnp.float32)
        h2 = jnp.maximum(a2 * dinv + b2_ref[...], 0.0)

        x3 = lax.dot_general(w3_ref[...], h2, (((0,), (0,)), ((), ())),
                             preferred_element_type=jnp.float32)
        a3 = jnp.dot(x3 * dinv, at, preferred_element_type=jnp.float32)
        logits = a3 * dinv + b3_ref[...]

        col = lax.broadcasted_iota(jnp.int32, (1, NUM), 1)
        ismain = col < MAIN
        mx = jnp.max(jnp.where(ismain, logits, -jnp.inf), axis=1, keepdims=True)
        e = jnp.exp(jnp.where(ismain, logits - mx, 0.0))
        ssum = jnp.sum(jnp.where(ismain, e, 0.0), axis=1, keepdims=True)
        sig = 1.0 / (1.0 + jnp.exp(-logits))
        out_ref[...] = jnp.where(ismain, e / ssum, sig)

    full = lambda shape: pl.BlockSpec(shape, lambda b: (0,) * len(shape))
    return pl.pallas_call(
        body,
        grid=(B,),
        in_specs=[
            pl.BlockSpec((1, NUM), lambda b: (b, 0)),
            full((NUM, D)),
            full((D, H)),
            full((H, 1)),
            full((H, H)),
            full((H, 1)),
            full((H, 1)),
            full((1, 1)),
            full((NUM, NUM)),
        ],
        out_specs=pl.BlockSpec((1, NUM), lambda b: (b, 0)),
        out_shape=jax.ShapeDtypeStruct((B, NUM), jnp.float32),
        scratch_shapes=[
            pltpu.VMEM((1, NUM), jnp.float32),
            pltpu.VMEM((H, NUM), jnp.float32),
        ],
        compiler_params=pltpu.CompilerParams(
            dimension_semantics=("arbitrary",)),
    )(x, embedding, W1, b1, W2, b2, W3, b3, at0)


def kernel(x, embedding, W1, b1, W2, b2, W3, b3, edge_index):
    src = edge_index[0]
    dst = edge_index[1]
    zeros = jnp.zeros((NUM * NUM,), jnp.float32)
    at0 = _sc_build_counts(src, dst, zeros).reshape(NUM, NUM)
    return _tc_gcn(x, embedding, W1, b1.reshape(H, 1), W2, b2.reshape(H, 1),
                   W3, b3.reshape(1, 1), at0)


# speedup vs baseline: 369.3384x; 369.3384x over previous
import functools

import jax
import jax.numpy as jnp
from jax import lax
from jax.experimental import pallas as pl
from jax.experimental.pallas import tpu as pltpu
import jax.experimental.pallas.tpu_sc as plsc

B = 64
NUM = 1024
D = 64
H = 64
E = 16384
MAIN = 16

NS = 16
EPT = E // NS
WPT = (NUM * NUM) // NS
NCHUNK = 8
CHUNK = EPT // NCHUNK


def _sc_build_counts(src, dst, zeros):
    mesh = plsc.VectorSubcoreMesh(core_axis_name="c", subcore_axis_name="s")

    idx_scratch = [pltpu.VMEM((CHUNK,), jnp.int32) for _ in range(NCHUNK)]

    @functools.partial(
        pl.kernel,
        mesh=mesh,
        out_type=jax.ShapeDtypeStruct((NUM * NUM,), jnp.float32),
        scratch_types=[
            pltpu.VMEM((EPT,), jnp.int32),
            pltpu.VMEM((EPT,), jnp.int32),
            pltpu.VMEM((CHUNK,), jnp.float32),
            pltpu.VMEM_SHARED((NUM * NUM,), jnp.float32),
        ] + idx_scratch,
    )
    def body(src_hbm, dst_hbm, zero_hbm, out_hbm, ev_src, ev_dst, ones_v,
             acc, *idx_refs):
        cid = lax.axis_index("c")
        sid = lax.axis_index("s")
        base = sid * WPT
        pltpu.sync_copy(zero_hbm.at[pl.ds(base, WPT)], acc.at[pl.ds(base, WPT)])
        ebase = sid * EPT
        pltpu.sync_copy(src_hbm.at[pl.ds(ebase, EPT)], ev_src)
        pltpu.sync_copy(dst_hbm.at[pl.ds(ebase, EPT)], ev_dst)
        ones16 = jnp.ones((16,), jnp.float32)
        for k in range(CHUNK // 16):
            ones_v[pl.ds(k * 16, 16)] = ones16
        for j in range(NCHUNK):
            for k in range(CHUNK // 16):
                off = j * CHUNK + k * 16
                s16 = ev_src[pl.ds(off, 16)]
                d16 = ev_dst[pl.ds(off, 16)]
                idx_refs[j][pl.ds(k * 16, 16)] = s16 * NUM + d16
        plsc.subcore_barrier()
        for j in range(NCHUNK):
            pltpu.sync_copy(ones_v, acc.at[idx_refs[j]], add=True)
        plsc.subcore_barrier()

        @pl.when(cid == 0)
        def _():
            pltpu.sync_copy(acc.at[pl.ds(base, WPT)],
                            out_hbm.at[pl.ds(base, WPT)])

    return body(src, dst, zeros)


def _tc_gcn(x, embedding, W1, b1, W2, b2, W3, b3, at0):
    def body(x_ref, emb_ref, w1_ref, b1_ref, w2_ref, b2_ref, w3_ref, b3_ref,
             at_ref, out_ref, dinv_ref, ew1_ref):
        b = pl.program_id(0)

        @pl.when(b == 0)
        def _():
            at0_full = at_ref[...]
            deg = jnp.sum(at0_full, axis=0, keepdims=True)
            dinv_ref[...] = jnp.where(
                deg > 0, lax.rsqrt(jnp.maximum(deg, 1e-12)), 0.0)
            ew1_ref[...] = lax.dot_general(
                w1_ref[...], emb_ref[...], (((0,), (1,)), ((), ())),
                preferred_element_type=jnp.float32)

        dinv = dinv_ref[...]
        at = at_ref[...]
        xb = x_ref[0]

        x1 = ew1_ref[...] * xb
        a1 = jnp.dot(x1 * dinv, at, preferred_element_type=jnp.float32)
        h1 = jnp.maximum(a1 * dinv + b1_ref[...], 0.0)

        x2 = lax.dot_general(w2_ref[...], h1, (((0,), (0,)), ((), ())),
                             preferred_element_type=jnp.float32)
        a2 = jnp.dot(x2 * dinv, at, preferred_element_type=jnp.float32)
        h2 = jnp.maximum(a2 * dinv + b2_ref[...], 0.0)

        x3 = lax.dot_general(w3_ref[...], h2, (((0,), (0,)), ((), ())),
                             preferred_element_type=jnp.float32)
        a3 = jnp.dot(x3 * dinv, at, preferred_element_type=jnp.float32)
        logits = a3 * dinv + b3_ref[...]

        col = lax.broadcasted_iota(jnp.int32, (1, NUM), 1)
        ismain = col < MAIN
        mx = jnp.max(jnp.where(ismain, logits, -jnp.inf), axis=1, keepdims=True)
        e = jnp.exp(jnp.where(ismain, logits - mx, 0.0))
        ssum = jnp.sum(jnp.where(ismain, e, 0.0), axis=1, keepdims=True)
        sig = 1.0 / (1.0 + jnp.exp(-logits))
        out_ref[0] = jnp.where(ismain, e / ssum, sig)

    full = lambda shape: pl.BlockSpec(shape, lambda b: (0,) * len(shape))
    return pl.pallas_call(
        body,
        grid=(B,),
        in_specs=[
            pl.BlockSpec((1, 1, NUM), lambda b: (b, 0, 0)),
            full((NUM, D)),
            full((D, H)),
            full((H, 1)),
            full((H, H)),
            full((H, 1)),
            full((H, 1)),
            full((1, 1)),
            full((NUM, NUM)),
        ],
        out_specs=pl.BlockSpec((1, 1, NUM), lambda b: (b, 0, 0)),
        out_shape=jax.ShapeDtypeStruct((B, 1, NUM), jnp.float32),
        scratch_shapes=[
            pltpu.VMEM((1, NUM), jnp.float32),
            pltpu.VMEM((H, NUM), jnp.float32),
        ],
        compiler_params=pltpu.CompilerParams(
            dimension_semantics=("arbitrary",)),
    )(x.reshape(B, 1, NUM), embedding, W1, b1, W2, b2, W3, b3,
      at0).reshape(B, NUM)


def kernel(x, embedding, W1, b1, W2, b2, W3, b3, edge_index):
    src = edge_index[0]
    dst = edge_index[1]
    zeros = jnp.zeros((NUM * NUM,), jnp.float32)
    at0 = _sc_build_counts(src, dst, zeros).reshape(NUM, NUM)
    return _tc_gcn(x, embedding, W1, b1.reshape(H, 1), W2, b2.reshape(H, 1),
                   W3, b3.reshape(1, 1), at0)


# speedup vs baseline: 716.2338x; 1.9392x over previous
import functools

import jax
import jax.numpy as jnp
from jax import lax
from jax.experimental import pallas as pl
from jax.experimental.pallas import tpu as pltpu
import jax.experimental.pallas.tpu_sc as plsc

B = 64
NUM = 1024
D = 64
H = 64
E = 16384
MAIN = 16

NS = 16
EPT = E // NS
WPT = (NUM * NUM) // NS
NCHUNK = 8
CHUNK = EPT // NCHUNK


def _sc_build_counts(src, dst, zeros):
    mesh = plsc.VectorSubcoreMesh(core_axis_name="c", subcore_axis_name="s")

    idx_scratch = [pltpu.VMEM((CHUNK,), jnp.int32) for _ in range(NCHUNK)]

    @functools.partial(
        pl.kernel,
        mesh=mesh,
        out_type=jax.ShapeDtypeStruct((NUM * NUM,), jnp.float32),
        scratch_types=[
            pltpu.VMEM((EPT,), jnp.int32),
            pltpu.VMEM((EPT,), jnp.int32),
            pltpu.VMEM((CHUNK,), jnp.float32),
            pltpu.VMEM_SHARED((NUM * NUM,), jnp.float32),
        ] + idx_scratch,
    )
    def body(src_hbm, dst_hbm, zero_hbm, out_hbm, ev_src, ev_dst, ones_v,
             acc, *idx_refs):
        cid = lax.axis_index("c")
        sid = lax.axis_index("s")
        base = sid * WPT
        pltpu.sync_copy(zero_hbm.at[pl.ds(base, WPT)], acc.at[pl.ds(base, WPT)])
        ebase = sid * EPT
        pltpu.sync_copy(src_hbm.at[pl.ds(ebase, EPT)], ev_src)
        pltpu.sync_copy(dst_hbm.at[pl.ds(ebase, EPT)], ev_dst)
        ones16 = jnp.ones((16,), jnp.float32)
        for k in range(CHUNK // 16):
            ones_v[pl.ds(k * 16, 16)] = ones16
        for j in range(NCHUNK):
            for k in range(CHUNK // 16):
                off = j * CHUNK + k * 16
                s16 = ev_src[pl.ds(off, 16)]
                d16 = ev_dst[pl.ds(off, 16)]
                idx_refs[j][pl.ds(k * 16, 16)] = s16 * NUM + d16
        plsc.subcore_barrier()
        for j in range(NCHUNK):
            pltpu.sync_copy(ones_v, acc.at[idx_refs[j]], add=True)
        plsc.subcore_barrier()

        @pl.when(cid == 0)
        def _():
            pltpu.sync_copy(acc.at[pl.ds(base, WPT)],
                            out_hbm.at[pl.ds(base, WPT)])

    return body(src, dst, zeros)


G = 8
GH = G * H


def _tc_gcn(x, embedding, W1, b1t, b2t, W2, W3, b3, at0):
    def body(x_ref, emb_ref, w1_ref, b1_ref, b2_ref, w2_ref, w3_ref, b3_ref,
             at_ref, out_ref, dinv_ref, ew1_ref):
        b = pl.program_id(0)

        @pl.when(b == 0)
        def _():
            at0_full = at_ref[...]
            deg = jnp.sum(at0_full, axis=0, keepdims=True)
            dinv_ref[...] = jnp.where(
                deg > 0, lax.rsqrt(jnp.maximum(deg, 1e-12)), 0.0)
            ew1_ref[...] = lax.dot_general(
                w1_ref[...], emb_ref[...], (((0,), (1,)), ((), ())),
                preferred_element_type=jnp.float32)

        dinv = dinv_ref[...]
        at = at_ref[...]
        xb = x_ref[...]

        x1 = (ew1_ref[...][:, None, :] * xb[None, :, :]).reshape(GH, NUM)
        a1 = jnp.dot(x1 * dinv, at, preferred_element_type=jnp.float32)
        h1 = jnp.maximum(a1 * dinv + b1_ref[...], 0.0)

        x2 = lax.dot_general(
            w2_ref[...], h1.reshape(H, G, NUM), (((0,), (0,)), ((), ())),
            preferred_element_type=jnp.float32).reshape(GH, NUM)
        a2 = jnp.dot(x2 * dinv, at, preferred_element_type=jnp.float32)
        h2 = jnp.maximum(a2 * dinv + b2_ref[...], 0.0)

        x3 = jnp.sum(h2.reshape(H, G, NUM) * w3_ref[...][:, :, None],
                     axis=0)
        a3 = jnp.dot(x3 * dinv, at, preferred_element_type=jnp.float32)
        logits = a3 * dinv + b3_ref[...]

        col = lax.broadcasted_iota(jnp.int32, (G, NUM), 1)
        ismain = col < MAIN
        mx = jnp.max(jnp.where(ismain, logits, -jnp.inf), axis=1, keepdims=True)
        e = jnp.exp(jnp.where(ismain, logits - mx, 0.0))
        ssum = jnp.sum(jnp.where(ismain, e, 0.0), axis=1, keepdims=True)
        sig = 1.0 / (1.0 + jnp.exp(-logits))
        out_ref[...] = jnp.where(ismain, e / ssum, sig)

    full = lambda shape: pl.BlockSpec(shape, lambda b: (0,) * len(shape))
    return pl.pallas_call(
        body,
        grid=(B // G,),
        in_specs=[
            pl.BlockSpec((G, NUM), lambda b: (b, 0)),
            full((NUM, D)),
            full((D, H)),
            full((GH, 1)),
            full((GH, 1)),
            full((H, H)),
            full((H, 1)),
            full((1, 1)),
            full((NUM, NUM)),
        ],
        out_specs=pl.BlockSpec((G, NUM), lambda b: (b, 0)),
        out_shape=jax.ShapeDtypeStruct((B, NUM), jnp.float32),
        scratch_shapes=[
            pltpu.VMEM((1, NUM), jnp.float32),
            pltpu.VMEM((H, NUM), jnp.float32),
        ],
        compiler_params=pltpu.CompilerParams(
            dimension_semantics=("arbitrary",)),
    )(x, embedding, W1, b1t, b2t, W2, W3, b3, at0)


def kernel(x, embedding, W1, b1, W2, b2, W3, b3, edge_index):
    src = edge_index[0]
    dst = edge_index[1]
    zeros = jnp.zeros((NUM * NUM,), jnp.float32)
    at0 = _sc_build_counts(src, dst, zeros).reshape(NUM, NUM)
    b1t = jnp.repeat(b1, G).reshape(GH, 1)
    b2t = jnp.repeat(b2, G).reshape(GH, 1)
    return _tc_gcn(x, embedding, W1, b1t, b2t, W2, W3, b3.reshape(1, 1), at0)


# speedup vs baseline: 735.5029x; 1.0269x over previous
import functools

import jax
import jax.numpy as jnp
from jax import lax
from jax.experimental import pallas as pl
from jax.experimental.pallas import tpu as pltpu
import jax.experimental.pallas.tpu_sc as plsc

B = 64
NUM = 1024
D = 64
H = 64
E = 16384
MAIN = 16

NS = 16
EPT = E // NS
WPT = (NUM * NUM) // NS
NCHUNK = 8
CHUNK = EPT // NCHUNK


def _sc_build_counts(src, dst, zeros):
    mesh = plsc.VectorSubcoreMesh(core_axis_name="c", subcore_axis_name="s")

    idx_scratch = [pltpu.VMEM((CHUNK,), jnp.int32) for _ in range(NCHUNK)]

    @functools.partial(
        pl.kernel,
        mesh=mesh,
        out_type=jax.ShapeDtypeStruct((NUM * NUM,), jnp.float32),
        scratch_types=[
            pltpu.VMEM((EPT,), jnp.int32),
            pltpu.VMEM((EPT,), jnp.int32),
            pltpu.VMEM((CHUNK,), jnp.float32),
            pltpu.VMEM_SHARED((NUM * NUM,), jnp.float32),
        ] + idx_scratch,
    )
    def body(src_hbm, dst_hbm, zero_hbm, out_hbm, ev_src, ev_dst, ones_v,
             acc, *idx_refs):
        cid = lax.axis_index("c")
        sid = lax.axis_index("s")
        base = sid * WPT
        pltpu.sync_copy(zero_hbm.at[pl.ds(base, WPT)], acc.at[pl.ds(base, WPT)])
        ebase = sid * EPT
        pltpu.sync_copy(src_hbm.at[pl.ds(ebase, EPT)], ev_src)
        pltpu.sync_copy(dst_hbm.at[pl.ds(ebase, EPT)], ev_dst)
        ones16 = jnp.ones((16,), jnp.float32)
        for k in range(CHUNK // 16):
            ones_v[pl.ds(k * 16, 16)] = ones16
        for j in range(NCHUNK):
            for k in range(CHUNK // 16):
                off = j * CHUNK + k * 16
                s16 = ev_src[pl.ds(off, 16)]
                d16 = ev_dst[pl.ds(off, 16)]
                idx_refs[j][pl.ds(k * 16, 16)] = s16 * NUM + d16
        plsc.subcore_barrier()
        for j in range(NCHUNK):
            pltpu.sync_copy(ones_v, acc.at[idx_refs[j]], add=True)
        plsc.subcore_barrier()

        @pl.when(cid == 0)
        def _():
            pltpu.sync_copy(acc.at[pl.ds(base, WPT)],
                            out_hbm.at[pl.ds(base, WPT)])

    return body(src, dst, zeros)


G = 16
GH = G * H


def _tc_gcn(x, embedding, W1, b1t, b2t, W2, W3, b3, at0):
    def body(x_ref, emb_ref, w1_ref, b1_ref, b2_ref, w2_ref, w3_ref, b3_ref,
             at_ref, out_ref, dinv_ref, ew1_ref):
        b = pl.program_id(0)

        @pl.when(b == 0)
        def _():
            at0_full = at_ref[...]
            deg = jnp.sum(at0_full, axis=0, keepdims=True)
            dinv_ref[...] = jnp.where(
                deg > 0, lax.rsqrt(jnp.maximum(deg, 1e-12)), 0.0)
            ew1_ref[...] = lax.dot_general(
                w1_ref[...], emb_ref[...], (((0,), (1,)), ((), ())),
                preferred_element_type=jnp.float32)

        dinv = dinv_ref[...]
        at = at_ref[...]
        xb = x_ref[...]

        x1 = (ew1_ref[...][:, None, :] * xb[None, :, :]).reshape(GH, NUM)
        a1 = jnp.dot(x1 * dinv, at, preferred_element_type=jnp.float32)
        h1 = jnp.maximum(a1 * dinv + b1_ref[...], 0.0)

        x2 = lax.dot_general(
            w2_ref[...], h1.reshape(H, G, NUM), (((0,), (0,)), ((), ())),
            preferred_element_type=jnp.float32).reshape(GH, NUM)
        a2 = jnp.dot(x2 * dinv, at, preferred_element_type=jnp.float32)
        h2 = jnp.maximum(a2 * dinv + b2_ref[...], 0.0)

        x3 = jnp.sum(h2.reshape(H, G, NUM) * w3_ref[...][:, :, None],
                     axis=0)
        a3 = jnp.dot(x3 * dinv, at, preferred_element_type=jnp.float32)
        logits = a3 * dinv + b3_ref[...]

        col = lax.broadcasted_iota(jnp.int32, (G, NUM), 1)
        ismain = col < MAIN
        mx = jnp.max(jnp.where(ismain, logits, -jnp.inf), axis=1, keepdims=True)
        e = jnp.exp(jnp.where(ismain, logits - mx, 0.0))
        ssum = jnp.sum(jnp.where(ismain, e, 0.0), axis=1, keepdims=True)
        sig = 1.0 / (1.0 + jnp.exp(-logits))
        out_ref[...] = jnp.where(ismain, e / ssum, sig)

    full = lambda shape: pl.BlockSpec(shape, lambda b: (0,) * len(shape))
    return pl.pallas_call(
        body,
        grid=(B // G,),
        in_specs=[
            pl.BlockSpec((G, NUM), lambda b: (b, 0)),
            full((NUM, D)),
            full((D, H)),
            full((GH, 1)),
            full((GH, 1)),
            full((H, H)),
            full((H, 1)),
            full((1, 1)),
            full((NUM, NUM)),
        ],
        out_specs=pl.BlockSpec((G, NUM), lambda b: (b, 0)),
        out_shape=jax.ShapeDtypeStruct((B, NUM), jnp.float32),
        scratch_shapes=[
            pltpu.VMEM((1, NUM), jnp.float32),
            pltpu.VMEM((H, NUM), jnp.float32),
        ],
        compiler_params=pltpu.CompilerParams(
            dimension_semantics=("arbitrary",)),
    )(x, embedding, W1, b1t, b2t, W2, W3, b3, at0)


def kernel(x, embedding, W1, b1, W2, b2, W3, b3, edge_index):
    src = edge_index[0]
    dst = edge_index[1]
    zeros = jnp.zeros((NUM * NUM,), jnp.float32)
    at0 = _sc_build_counts(src, dst, zeros).reshape(NUM, NUM)
    b1t = jnp.repeat(b1, G).reshape(GH, 1)
    b2t = jnp.repeat(b2, G).reshape(GH, 1)
    return _tc_gcn(x, embedding, W1, b1t, b2t, W2, W3, b3.reshape(1, 1), at0)
